# Initial kernel scaffold; baseline (speedup 1.0000x reference)
import functools

import jax
import jax.numpy as jnp
from jax import lax
from jax.experimental import pallas as pl
from jax.experimental.pallas import tpu as pltpu
from jax.experimental.pallas import tpu_sc as plsc

_B, _C, _N_IN = 4, 64, 196608
_N_OUT = _N_IN // 4
_TOT_IN = _B * _C * _N_IN
_TOT_OUT = _TOT_IN // 4
_NC, _NS = 2, 16
_NW = _NC * _NS
_IN_PER_W = _TOT_IN // _NW
_OUT_PER_W = _IN_PER_W // 4
_IC = 49152
_OC = _IC // 4
_NCHUNK = _IN_PER_W // _IC


@functools.partial(
    pl.kernel,
    out_type=jax.ShapeDtypeStruct((_TOT_OUT,), jnp.float32),
    mesh=plsc.VectorSubcoreMesh(
        core_axis_name="c", subcore_axis_name="s",
        num_cores=_NC, num_subcores=_NS),
    scratch_types=[
        pltpu.VMEM((2, _IC), jnp.float32),
        pltpu.VMEM((2, _OC), jnp.float32),
        pltpu.SemaphoreType.DMA((2,)),
        pltpu.SemaphoreType.DMA((2,)),
    ],
)
def _down(x_hbm, y_hbm, ibuf, obuf, isem, osem):
    cid = lax.axis_index("c")
    sid = lax.axis_index("s")
    wid = sid * _NC + cid
    in_base = wid * _IN_PER_W
    out_base = wid * _OUT_PER_W
    iota4 = lax.iota(jnp.int32, 16) * 4

    pltpu.async_copy(x_hbm.at[pl.ds(in_base, _IC)], ibuf.at[0], isem.at[0])

    @pl.loop(0, _NCHUNK // 2)
    def _outer(gg):
        for b in range(2):
            g = gg * 2 + b
            nxt = 1 - b

            @pl.when(g + 1 < _NCHUNK)
            def _():
                pltpu.async_copy(
                    x_hbm.at[pl.ds(in_base + (g + 1) * _IC, _IC)],
                    ibuf.at[nxt], isem.at[nxt])

            pltpu.make_async_copy(
                x_hbm.at[pl.ds(in_base, _IC)], ibuf.at[b], isem.at[b]).wait()

            @pl.when(g >= 2)
            def _():
                pltpu.make_async_copy(
                    obuf.at[b], y_hbm.at[pl.ds(out_base, _OC)],
                    osem.at[b]).wait()

            ib = ibuf.at[b]

            @pl.loop(0, _OC // 16, unroll=8)
            def _step(v):
                i0 = v * 64 + iota4
                v0 = plsc.load_gather(ib, [i0])
                v1 = plsc.load_gather(ib, [i0 + 1])
                v2 = plsc.load_gather(ib, [i0 + 2])
                v3 = plsc.load_gather(ib, [i0 + 3])
                m = jnp.maximum(jnp.maximum(v0, v1), jnp.maximum(v2, v3))
                obuf[b, pl.ds(v * 16, 16)] = m

            pltpu.async_copy(
                obuf.at[b], y_hbm.at[pl.ds(out_base + g * _OC, _OC)],
                osem.at[b])

    for b in range(2):
        pltpu.make_async_copy(
            obuf.at[b], y_hbm.at[pl.ds(out_base, _OC)], osem.at[b]).wait()


def kernel(x):
    y_flat = _down(x.reshape(-1))
    return y_flat.reshape(_B, _C, _N_OUT)


# speedup vs baseline: 2.4810x; 2.4810x over previous
import functools

import jax
import jax.numpy as jnp
from jax import lax
from jax.experimental import pallas as pl
from jax.experimental.pallas import tpu as pltpu
from jax.experimental.pallas import tpu_sc as plsc

_B, _C, _N_IN = 4, 64, 196608
_N_OUT = _N_IN // 4
_TOT_IN = _B * _C * _N_IN
_TOT_OUT = _TOT_IN // 4
_NC, _NS = 2, 16
_NW = _NC * _NS
_IN_PER_W = _TOT_IN // _NW
_OUT_PER_W = _IN_PER_W // 4
_IC = 49152
_OC = _IC // 4
_NCHUNK = _IN_PER_W // _IC


@functools.partial(
    pl.kernel,
    out_type=jax.ShapeDtypeStruct((_TOT_OUT,), jnp.float32),
    mesh=plsc.VectorSubcoreMesh(
        core_axis_name="c", subcore_axis_name="s",
        num_cores=_NC, num_subcores=_NS),
    scratch_types=[
        pltpu.VMEM((_IC,), jnp.float32),
        pltpu.VMEM((_IC,), jnp.float32),
        pltpu.VMEM((_OC,), jnp.float32),
        pltpu.VMEM((_OC,), jnp.float32),
        pltpu.SemaphoreType.DMA((2,)),
        pltpu.SemaphoreType.DMA((2,)),
    ],
    compiler_params=pltpu.CompilerParams(needs_layout_passes=False),
)
def _down(x_hbm, y_hbm, ibuf0, ibuf1, obuf0, obuf1, isem, osem):
    ibuf = (ibuf0, ibuf1)
    obuf = (obuf0, obuf1)
    cid = lax.axis_index("c")
    sid = lax.axis_index("s")
    wid = sid * _NC + cid
    in_base = wid * _IN_PER_W
    out_base = wid * _OUT_PER_W
    iota4 = lax.iota(jnp.int32, 16) * 4

    pltpu.async_copy(x_hbm.at[pl.ds(in_base, _IC)], ibuf[0], isem.at[0])

    @pl.loop(0, _NCHUNK // 2)
    def _outer(gg):
        for b in range(2):
            g = gg * 2 + b
            nxt = 1 - b

            @pl.when(g + 1 < _NCHUNK)
            def _():
                pltpu.async_copy(
                    x_hbm.at[pl.ds(in_base + (g + 1) * _IC, _IC)],
                    ibuf[nxt], isem.at[nxt])

            pltpu.make_async_copy(
                x_hbm.at[pl.ds(in_base, _IC)], ibuf[b], isem.at[b]).wait()

            @pl.when(g >= 2)
            def _():
                pltpu.make_async_copy(
                    obuf[b], y_hbm.at[pl.ds(out_base, _OC)],
                    osem.at[b]).wait()

            ib = ibuf[b]
            ob = obuf[b]

            @pl.loop(0, _OC // 16, unroll=8)
            def _step(v):
                i0 = v * 64 + iota4
                v0 = plsc.load_gather(ib, [i0])
                v1 = plsc.load_gather(ib, [i0 + 1])
                v2 = plsc.load_gather(ib, [i0 + 2])
                v3 = plsc.load_gather(ib, [i0 + 3])
                m = jnp.maximum(jnp.maximum(v0, v1), jnp.maximum(v2, v3))
                ob[pl.ds(v * 16, 16)] = m

            pltpu.async_copy(
                obuf[b], y_hbm.at[pl.ds(out_base + g * _OC, _OC)],
                osem.at[b])

    for b in range(2):
        pltpu.make_async_copy(
            obuf[b], y_hbm.at[pl.ds(out_base, _OC)], osem.at[b]).wait()


def kernel(x):
    y_flat = _down(x.reshape(-1))
    return y_flat.reshape(_B, _C, _N_OUT)
